# baseline (speedup 1.0000x reference)
import functools

import jax
import jax.numpy as jnp
from jax import lax
from jax.experimental import pallas as pl
from jax.experimental.pallas import tpu as pltpu
from jax.experimental.pallas import tpu_sc as plsc

N = 100000
E = 6400000

NC = 2
NS = 16
NW = NC * NS

NP = 102400
ZR = NP // NS
NPR = NP // 128
D = 8

K = 8
RB = E // (K * 128)
NCH_BASE = RB // NW
NCH_EXTRA = RB % NW

_MESH = plsc.VectorSubcoreMesh(
    core_axis_name="c", subcore_axis_name="s", num_cores=NC, num_subcores=NS
)
_SC_PARAMS = pltpu.CompilerParams(use_tc_tiling_on_sc=False)


def _drain_chunk(zeros_hbm, dummy_dst, sem):
    for _ in range(K):
        pltpu.make_async_copy(zeros_hbm.at[pl.ds(0, 128), :], dummy_dst, sem).wait()


@functools.partial(
    pl.kernel,
    out_type=jax.ShapeDtypeStruct((NC, NP, D), jnp.float32),
    mesh=_MESH,
    scratch_types=[
        pltpu.VMEM((2 * K, 128), jnp.int32),
        pltpu.VMEM((128, D), jnp.float32),
        pltpu.VMEM_SHARED((NP, D), jnp.float32),
        pltpu.SemaphoreType.DMA,
    ],
    compiler_params=_SC_PARAMS,
)
def _sc_degree(edge_hbm, ones_hbm, zeros_hbm, out_hbm, didx, ones_v, acc, sem):
    c = lax.axis_index("c")
    s = lax.axis_index("s")
    wid = s * NC + c

    pltpu.sync_copy(ones_hbm, ones_v)
    pltpu.sync_copy(zeros_hbm.at[pl.ds(s * ZR, ZR), :], acc.at[pl.ds(s * ZR, ZR), :])
    plsc.subcore_barrier()

    chunk0 = NCH_BASE * wid + jnp.minimum(wid, NCH_EXTRA)
    nch = NCH_BASE + jnp.where(wid < NCH_EXTRA, 1, 0)

    def fire(i, slot):
        for j in range(K):
            pltpu.async_copy(ones_v, acc.at[didx.at[slot * K + j]], sem, add=True)

    pltpu.sync_copy(edge_hbm.at[1, chunk0], didx.at[pl.ds(0, K)])
    fire(0, 0)
    pltpu.sync_copy(edge_hbm.at[1, chunk0 + 1], didx.at[pl.ds(K, K)])
    fire(1, 1)

    def body(i, carry):
        p = lax.rem(i, 2)
        _drain_chunk(zeros_hbm, ones_v, sem)
        pltpu.sync_copy(edge_hbm.at[1, chunk0 + i], didx.at[pl.ds(p * K, K)])
        fire(i, p)
        return carry

    lax.fori_loop(2, nch, body, 0)
    _drain_chunk(zeros_hbm, ones_v, sem)
    _drain_chunk(zeros_hbm, ones_v, sem)
    plsc.subcore_barrier()
    pltpu.sync_copy(acc.at[pl.ds(s * ZR, ZR), :], out_hbm.at[c, pl.ds(s * ZR, ZR), :])


@functools.partial(
    pl.kernel,
    out_type=jax.ShapeDtypeStruct((NC, NP, D), jnp.float32),
    mesh=_MESH,
    scratch_types=[
        pltpu.VMEM((2 * K, 128), jnp.int32),
        pltpu.VMEM((2 * K, 128), jnp.int32),
        pltpu.VMEM((2 * K, 128, D), jnp.float32),
        pltpu.VMEM_SHARED((NP, D), jnp.float32),
        pltpu.VMEM_SHARED((NP, D), jnp.float32),
        pltpu.SemaphoreType.DMA,
        pltpu.SemaphoreType.DMA,
    ],
    compiler_params=_SC_PARAMS,
)
def _sc_aggregate(
    g_hbm, edge_hbm, zeros_hbm, out_hbm,
    sidx, didx, rows, tabl, acc, sem_g, sem_s,
):
    c = lax.axis_index("c")
    s = lax.axis_index("s")
    wid = s * NC + c

    pltpu.sync_copy(zeros_hbm.at[pl.ds(s * ZR, ZR), :], acc.at[pl.ds(s * ZR, ZR), :])
    plsc.subcore_barrier()

    chunk0 = NCH_BASE * wid + jnp.minimum(wid, NCH_EXTRA)
    nch = NCH_BASE + jnp.where(wid < NCH_EXTRA, 1, 0)
    dummy = rows.at[0]

    def load_idx(i, slot):
        pltpu.sync_copy(edge_hbm.at[0, chunk0 + i], sidx.at[pl.ds(slot * K, K)])
        pltpu.sync_copy(edge_hbm.at[1, chunk0 + i], didx.at[pl.ds(slot * K, K)])

    def fire_gathers(slot):
        for j in range(K):
            pltpu.async_copy(
                g_hbm.at[sidx.at[slot * K + j]], rows.at[slot * K + j], sem_g
            )

    def fire_scatters(slot):
        for j in range(K):
            pltpu.async_copy(
                rows.at[slot * K + j], acc.at[didx.at[slot * K + j]], sem_s,
                add=True,
            )

    load_idx(0, 0)
    fire_gathers(0)
    load_idx(1, 1)
    fire_gathers(1)
    _drain_chunk(zeros_hbm, dummy, sem_g)
    fire_scatters(0)

    def body(i, carry):
        p = lax.rem(i, 2)
        _drain_chunk(zeros_hbm, dummy, sem_s)
        _drain_chunk(zeros_hbm, dummy, sem_g)
        load_idx(i, p)
        fire_gathers(p)
        fire_scatters(1 - p)
        return carry

    lax.fori_loop(2, nch, body, 0)
    _drain_chunk(zeros_hbm, dummy, sem_g)
    fire_scatters(lax.rem(nch - 1, 2))
    _drain_chunk(zeros_hbm, dummy, sem_s)
    _drain_chunk(zeros_hbm, dummy, sem_s)
    plsc.subcore_barrier()
    pltpu.sync_copy(
        acc.at[pl.ds(s * ZR, ZR), :], out_hbm.at[c, pl.ds(s * ZR, ZR), :]
    )


def _glue_a_body(degp_ref, xpl_ref, dinv_ref, g_ref):
    d = degp_ref[...]
    deg = d[0] + d[1] + 1.0
    dinv = lax.rsqrt(deg)
    dinv_ref[...] = dinv
    g_ref[...] = xpl_ref[...] * dinv[None]


def _glue_b_body(sp_ref, g_ref, dinv_ref, w1_ref, b1_ref, w2_ref, out_ref):
    sp = sp_ref[...]
    g = g_ref[...]
    dv = dinv_ref[...]
    y0 = dv * (sp[0, 0] + sp[1, 0] + g[0])
    y1 = dv * (sp[0, 1] + sp[1, 1] + g[1])
    w1 = w1_ref[...]
    b1 = b1_ref[...]
    w2 = w2_ref[...]
    z0 = jnp.zeros_like(y0)
    z1 = jnp.zeros_like(y0)
    for j in range(16):
        h = jnp.maximum(y0 * w1[0, j] + y1 * w1[1, j] + b1[0, j], 0.0)
        z0 = z0 + h * w2[j, 0]
        z1 = z1 + h * w2[j, 1]
    out_ref[...] = jnp.stack([z0 * dv, z1 * dv], axis=0)


def _glue_c_body(sp_ref, g_ref, dinv_ref, b2_ref, out_ref):
    sp = sp_ref[...]
    g = g_ref[...]
    dv = dinv_ref[...]
    b2 = b2_ref[...]
    y0 = dv * (sp[0, 0] + sp[1, 0] + g[0]) + b2[0, 0]
    y1 = dv * (sp[0, 1] + sp[1, 1] + g[1]) + b2[0, 1]
    out_ref[...] = jnp.stack([y0, y1], axis=0)


_PLANAR = jax.ShapeDtypeStruct((2, NPR, 128), jnp.float32)

_glue_a = pl.pallas_call(
    _glue_a_body,
    out_shape=(jax.ShapeDtypeStruct((NPR, 128), jnp.float32), _PLANAR),
)
_glue_b = pl.pallas_call(_glue_b_body, out_shape=_PLANAR)
_glue_c = pl.pallas_call(_glue_c_body, out_shape=_PLANAR)


def _widen(g2):
    return jnp.pad(g2.reshape(2, NP).T, ((0, 0), (0, D - 2)))


def _parts(s):
    return s[:, :, :2].transpose(0, 2, 1).reshape(NC, 2, NPR, 128)


def kernel(x, edge_index, W1, b1, W2, b2):
    ei = edge_index.reshape(2, RB, K, 128)

    zeros8 = jnp.zeros((NP, D), jnp.float32)
    ones8 = jnp.ones((128, D), jnp.float32)

    deg_parts = _sc_degree(ei, ones8, zeros8)
    degp = deg_parts[:, :, 0].reshape(NC, NPR, 128)

    xp = jnp.pad(x, ((0, NP - N), (0, 0)))
    xpl = xp.T.reshape(2, NPR, 128)
    dinv, g1 = _glue_a(degp, xpl)

    s1 = _sc_aggregate(_widen(g1), ei, zeros8)
    g2 = _glue_b(_parts(s1), g1, dinv, W1, b1.reshape(1, 16), W2)
    s2 = _sc_aggregate(_widen(g2), ei, zeros8)
    outp = _glue_c(_parts(s2), g2, dinv, b2.reshape(1, 2))
    return outp.reshape(2, NP).T[:N]


# speedup vs baseline: 1.3657x; 1.3657x over previous
import functools

import jax
import jax.numpy as jnp
from jax import lax
from jax.experimental import pallas as pl
from jax.experimental.pallas import tpu as pltpu
from jax.experimental.pallas import tpu_sc as plsc

N = 100000
E = 6400000

NC = 2
NS = 16
NW = NC * NS

NP = 102400
ZR = NP // NS
NPR = NP // 128
D = 8

K = 10
RB = E // (K * 128)
NCH_BASE = RB // NW
NCH_EXTRA = RB % NW

_MESH = plsc.VectorSubcoreMesh(
    core_axis_name="c", subcore_axis_name="s", num_cores=NC, num_subcores=NS
)
_SC_PARAMS = pltpu.CompilerParams(use_tc_tiling_on_sc=False)


def _drain_chunk(zeros_hbm, dummy_dst, sem):
    for _ in range(K):
        pltpu.make_async_copy(zeros_hbm.at[pl.ds(0, 128), :], dummy_dst, sem).wait()


@functools.partial(
    pl.kernel,
    out_type=jax.ShapeDtypeStruct((NC, NP, D), jnp.float32),
    mesh=_MESH,
    scratch_types=[
        pltpu.VMEM((2 * K, 128), jnp.int32),
        pltpu.VMEM((128, D), jnp.float32),
        pltpu.VMEM_SHARED((NP, D), jnp.float32),
        pltpu.SemaphoreType.DMA,
    ],
    compiler_params=_SC_PARAMS,
)
def _sc_degree(edge_hbm, ones_hbm, zeros_hbm, out_hbm, didx, ones_v, acc, sem):
    c = lax.axis_index("c")
    s = lax.axis_index("s")
    wid = s * NC + c

    pltpu.sync_copy(ones_hbm, ones_v)
    pltpu.sync_copy(zeros_hbm.at[pl.ds(s * ZR, ZR), :], acc.at[pl.ds(s * ZR, ZR), :])
    plsc.subcore_barrier()

    chunk0 = NCH_BASE * wid + jnp.minimum(wid, NCH_EXTRA)
    nch = NCH_BASE + jnp.where(wid < NCH_EXTRA, 1, 0)

    def fire(i, slot):
        for j in range(K):
            pltpu.async_copy(ones_v, acc.at[didx.at[slot * K + j]], sem, add=True)

    pltpu.sync_copy(edge_hbm.at[1, chunk0], didx.at[pl.ds(0, K)])
    fire(0, 0)
    pltpu.sync_copy(edge_hbm.at[1, chunk0 + 1], didx.at[pl.ds(K, K)])
    fire(1, 1)

    def body(i, carry):
        p = lax.rem(i, 2)
        _drain_chunk(zeros_hbm, ones_v, sem)
        pltpu.sync_copy(edge_hbm.at[1, chunk0 + i], didx.at[pl.ds(p * K, K)])
        fire(i, p)
        return carry

    lax.fori_loop(2, nch, body, 0)
    _drain_chunk(zeros_hbm, ones_v, sem)
    _drain_chunk(zeros_hbm, ones_v, sem)
    plsc.subcore_barrier()
    pltpu.sync_copy(acc.at[pl.ds(s * ZR, ZR), :], out_hbm.at[c, pl.ds(s * ZR, ZR), :])


@functools.partial(
    pl.kernel,
    out_type=jax.ShapeDtypeStruct((NC, NP, D), jnp.float32),
    mesh=_MESH,
    scratch_types=[
        pltpu.VMEM((2 * K, 128), jnp.int32),
        pltpu.VMEM((2 * K, 128), jnp.int32),
        pltpu.VMEM((2 * K, 128, D), jnp.float32),
        pltpu.VMEM_SHARED((NP, D), jnp.float32),
        pltpu.VMEM_SHARED((NP, D), jnp.float32),
        pltpu.SemaphoreType.DMA,
        pltpu.SemaphoreType.DMA,
    ],
    compiler_params=_SC_PARAMS,
)
def _sc_aggregate(
    g_hbm, edge_hbm, zeros_hbm, out_hbm,
    sidx, didx, rows, tabl, acc, sem_g, sem_s,
):
    c = lax.axis_index("c")
    s = lax.axis_index("s")
    wid = s * NC + c

    pltpu.sync_copy(g_hbm.at[pl.ds(s * ZR, ZR), :], tabl.at[pl.ds(s * ZR, ZR), :])
    pltpu.sync_copy(zeros_hbm.at[pl.ds(s * ZR, ZR), :], acc.at[pl.ds(s * ZR, ZR), :])
    plsc.subcore_barrier()

    chunk0 = NCH_BASE * wid + jnp.minimum(wid, NCH_EXTRA)
    nch = NCH_BASE + jnp.where(wid < NCH_EXTRA, 1, 0)
    dummy = rows.at[0]

    def load_idx(i, slot):
        pltpu.sync_copy(edge_hbm.at[0, chunk0 + i], sidx.at[pl.ds(slot * K, K)])
        pltpu.sync_copy(edge_hbm.at[1, chunk0 + i], didx.at[pl.ds(slot * K, K)])

    def fire_gathers(slot):
        for j in range(K):
            pltpu.async_copy(
                tabl.at[sidx.at[slot * K + j]], rows.at[slot * K + j], sem_g
            )

    def fire_scatters(slot):
        for j in range(K):
            pltpu.async_copy(
                rows.at[slot * K + j], acc.at[didx.at[slot * K + j]], sem_s,
                add=True,
            )

    load_idx(0, 0)
    fire_gathers(0)
    load_idx(1, 1)
    fire_gathers(1)
    _drain_chunk(zeros_hbm, dummy, sem_g)
    fire_scatters(0)

    def body(i, carry):
        p = lax.rem(i, 2)
        _drain_chunk(zeros_hbm, dummy, sem_s)
        _drain_chunk(zeros_hbm, dummy, sem_g)
        load_idx(i, p)
        fire_gathers(p)
        fire_scatters(1 - p)
        return carry

    lax.fori_loop(2, nch, body, 0)
    _drain_chunk(zeros_hbm, dummy, sem_g)
    fire_scatters(lax.rem(nch - 1, 2))
    _drain_chunk(zeros_hbm, dummy, sem_s)
    _drain_chunk(zeros_hbm, dummy, sem_s)
    plsc.subcore_barrier()
    pltpu.sync_copy(
        acc.at[pl.ds(s * ZR, ZR), :], out_hbm.at[c, pl.ds(s * ZR, ZR), :]
    )


def _glue_a_body(degp_ref, xpl_ref, dinv_ref, g_ref):
    d = degp_ref[...]
    deg = d[0] + d[1] + 1.0
    dinv = lax.rsqrt(deg)
    dinv_ref[...] = dinv
    g_ref[...] = xpl_ref[...] * dinv[None]


def _glue_b_body(sp_ref, g_ref, dinv_ref, w1_ref, b1_ref, w2_ref, out_ref):
    sp = sp_ref[...]
    g = g_ref[...]
    dv = dinv_ref[...]
    y0 = dv * (sp[0, 0] + sp[1, 0] + g[0])
    y1 = dv * (sp[0, 1] + sp[1, 1] + g[1])
    w1 = w1_ref[...]
    b1 = b1_ref[...]
    w2 = w2_ref[...]
    z0 = jnp.zeros_like(y0)
    z1 = jnp.zeros_like(y0)
    for j in range(16):
        h = jnp.maximum(y0 * w1[0, j] + y1 * w1[1, j] + b1[0, j], 0.0)
        z0 = z0 + h * w2[j, 0]
        z1 = z1 + h * w2[j, 1]
    out_ref[...] = jnp.stack([z0 * dv, z1 * dv], axis=0)


def _glue_c_body(sp_ref, g_ref, dinv_ref, b2_ref, out_ref):
    sp = sp_ref[...]
    g = g_ref[...]
    dv = dinv_ref[...]
    b2 = b2_ref[...]
    y0 = dv * (sp[0, 0] + sp[1, 0] + g[0]) + b2[0, 0]
    y1 = dv * (sp[0, 1] + sp[1, 1] + g[1]) + b2[0, 1]
    out_ref[...] = jnp.stack([y0, y1], axis=0)


_PLANAR = jax.ShapeDtypeStruct((2, NPR, 128), jnp.float32)

_glue_a = pl.pallas_call(
    _glue_a_body,
    out_shape=(jax.ShapeDtypeStruct((NPR, 128), jnp.float32), _PLANAR),
)
_glue_b = pl.pallas_call(_glue_b_body, out_shape=_PLANAR)
_glue_c = pl.pallas_call(_glue_c_body, out_shape=_PLANAR)


def _widen(g2):
    return jnp.pad(g2.reshape(2, NP).T, ((0, 0), (0, D - 2)))


def _parts(s):
    return s[:, :, :2].transpose(0, 2, 1).reshape(NC, 2, NPR, 128)


def kernel(x, edge_index, W1, b1, W2, b2):
    ei = edge_index.reshape(2, RB, K, 128)

    zeros8 = jnp.zeros((NP, D), jnp.float32)
    ones8 = jnp.ones((128, D), jnp.float32)

    deg_parts = _sc_degree(ei, ones8, zeros8)
    degp = deg_parts[:, :, 0].reshape(NC, NPR, 128)

    xp = jnp.pad(x, ((0, NP - N), (0, 0)))
    xpl = xp.T.reshape(2, NPR, 128)
    dinv, g1 = _glue_a(degp, xpl)

    s1 = _sc_aggregate(_widen(g1), ei, zeros8)
    g2 = _glue_b(_parts(s1), g1, dinv, W1, b1.reshape(1, 16), W2)
    s2 = _sc_aggregate(_widen(g2), ei, zeros8)
    outp = _glue_c(_parts(s2), g2, dinv, b2.reshape(1, 2))
    return outp.reshape(2, NP).T[:N]


# speedup vs baseline: 1.3781x; 1.0091x over previous
import functools

import jax
import jax.numpy as jnp
from jax import lax
from jax.experimental import pallas as pl
from jax.experimental.pallas import tpu as pltpu
from jax.experimental.pallas import tpu_sc as plsc

N = 100000
E = 6400000

NC = 2
NS = 16
NW = NC * NS

NP = 100096
ZR = NP // NS
NPR = NP // 128
D = 8

K = 8
RB = E // (K * 128)
NCH_BASE = RB // NW
NCH_EXTRA = RB % NW

_MESH = plsc.VectorSubcoreMesh(
    core_axis_name="c", subcore_axis_name="s", num_cores=NC, num_subcores=NS
)
_SC_PARAMS = pltpu.CompilerParams(use_tc_tiling_on_sc=False)


def _drain_chunk(zeros_hbm, dummy_dst, sem):
    for _ in range(K):
        pltpu.make_async_copy(zeros_hbm.at[pl.ds(0, 128), :], dummy_dst, sem).wait()


@functools.partial(
    pl.kernel,
    out_type=jax.ShapeDtypeStruct((NC, NP, D), jnp.float32),
    mesh=_MESH,
    scratch_types=[
        pltpu.VMEM((2 * K, 128), jnp.int32),
        pltpu.VMEM((128, D), jnp.float32),
        pltpu.VMEM_SHARED((NP, D), jnp.float32),
        pltpu.SemaphoreType.DMA,
    ],
    compiler_params=_SC_PARAMS,
)
def _sc_degree(edge_hbm, ones_hbm, zeros_hbm, out_hbm, didx, ones_v, acc, sem):
    c = lax.axis_index("c")
    s = lax.axis_index("s")
    wid = s * NC + c

    pltpu.sync_copy(ones_hbm, ones_v)
    pltpu.sync_copy(zeros_hbm.at[pl.ds(s * ZR, ZR), :], acc.at[pl.ds(s * ZR, ZR), :])
    plsc.subcore_barrier()

    chunk0 = NCH_BASE * wid + jnp.minimum(wid, NCH_EXTRA)
    nch = NCH_BASE + jnp.where(wid < NCH_EXTRA, 1, 0)

    def fire(i, slot):
        for j in range(K):
            pltpu.async_copy(ones_v, acc.at[didx.at[slot * K + j]], sem, add=True)

    pltpu.sync_copy(edge_hbm.at[1, chunk0], didx.at[pl.ds(0, K)])
    fire(0, 0)
    pltpu.sync_copy(edge_hbm.at[1, chunk0 + 1], didx.at[pl.ds(K, K)])
    fire(1, 1)

    def body(i, carry):
        p = lax.rem(i, 2)
        _drain_chunk(zeros_hbm, ones_v, sem)
        pltpu.sync_copy(edge_hbm.at[1, chunk0 + i], didx.at[pl.ds(p * K, K)])
        fire(i, p)
        return carry

    lax.fori_loop(2, nch, body, 0)
    _drain_chunk(zeros_hbm, ones_v, sem)
    _drain_chunk(zeros_hbm, ones_v, sem)
    plsc.subcore_barrier()
    pltpu.sync_copy(acc.at[pl.ds(s * ZR, ZR), :], out_hbm.at[c, pl.ds(s * ZR, ZR), :])


@functools.partial(
    pl.kernel,
    out_type=jax.ShapeDtypeStruct((NC, NP, D), jnp.float32),
    mesh=_MESH,
    scratch_types=[
        pltpu.VMEM((3 * K, 128), jnp.int32),
        pltpu.VMEM((3 * K, 128), jnp.int32),
        pltpu.VMEM((3 * K, 128, D), jnp.float32),
        pltpu.VMEM_SHARED((NP, D), jnp.float32),
        pltpu.VMEM_SHARED((NP, D), jnp.float32),
        pltpu.SemaphoreType.DMA,
        pltpu.SemaphoreType.DMA,
    ],
    compiler_params=_SC_PARAMS,
)
def _sc_aggregate(
    g_hbm, edge_hbm, zeros_hbm, out_hbm,
    sidx, didx, rows, tabl, acc, sem_g, sem_s,
):
    c = lax.axis_index("c")
    s = lax.axis_index("s")
    wid = s * NC + c

    pltpu.sync_copy(g_hbm.at[pl.ds(s * ZR, ZR), :], tabl.at[pl.ds(s * ZR, ZR), :])
    pltpu.sync_copy(zeros_hbm.at[pl.ds(s * ZR, ZR), :], acc.at[pl.ds(s * ZR, ZR), :])
    plsc.subcore_barrier()

    chunk0 = NCH_BASE * wid + jnp.minimum(wid, NCH_EXTRA)
    nch = NCH_BASE + jnp.where(wid < NCH_EXTRA, 1, 0)
    dummy = rows.at[0]

    def load_idx(i, slot):
        pltpu.sync_copy(edge_hbm.at[0, chunk0 + i], sidx.at[pl.ds(slot * K, K)])
        pltpu.sync_copy(edge_hbm.at[1, chunk0 + i], didx.at[pl.ds(slot * K, K)])

    def fire_gathers(slot):
        for j in range(K):
            pltpu.async_copy(
                tabl.at[sidx.at[slot * K + j]], rows.at[slot * K + j], sem_g
            )

    def fire_scatters(slot):
        for j in range(K):
            pltpu.async_copy(
                rows.at[slot * K + j], acc.at[didx.at[slot * K + j]], sem_s,
                add=True,
            )

    load_idx(0, 0)
    fire_gathers(0)
    load_idx(1, 1)
    fire_gathers(1)
    load_idx(2, 2)
    fire_gathers(2)
    _drain_chunk(zeros_hbm, dummy, sem_g)
    fire_scatters(0)
    _drain_chunk(zeros_hbm, dummy, sem_g)
    fire_scatters(1)

    def body(i, carry):
        p = lax.rem(i, 3)
        _drain_chunk(zeros_hbm, dummy, sem_s)
        _drain_chunk(zeros_hbm, dummy, sem_g)
        load_idx(i, p)
        fire_gathers(p)
        fire_scatters(lax.rem(i - 1, 3))
        return carry

    lax.fori_loop(3, nch, body, 0)
    _drain_chunk(zeros_hbm, dummy, sem_g)
    fire_scatters(lax.rem(nch - 1, 3))
    _drain_chunk(zeros_hbm, dummy, sem_s)
    _drain_chunk(zeros_hbm, dummy, sem_s)
    _drain_chunk(zeros_hbm, dummy, sem_s)
    plsc.subcore_barrier()
    pltpu.sync_copy(
        acc.at[pl.ds(s * ZR, ZR), :], out_hbm.at[c, pl.ds(s * ZR, ZR), :]
    )


def _glue_a_body(degp_ref, xpl_ref, dinv_ref, g_ref):
    d = degp_ref[...]
    deg = d[0] + d[1] + 1.0
    dinv = lax.rsqrt(deg)
    dinv_ref[...] = dinv
    g_ref[...] = xpl_ref[...] * dinv[None]


def _glue_b_body(sp_ref, g_ref, dinv_ref, w1_ref, b1_ref, w2_ref, out_ref):
    sp = sp_ref[...]
    g = g_ref[...]
    dv = dinv_ref[...]
    y0 = dv * (sp[0, 0] + sp[1, 0] + g[0])
    y1 = dv * (sp[0, 1] + sp[1, 1] + g[1])
    w1 = w1_ref[...]
    b1 = b1_ref[...]
    w2 = w2_ref[...]
    z0 = jnp.zeros_like(y0)
    z1 = jnp.zeros_like(y0)
    for j in range(16):
        h = jnp.maximum(y0 * w1[0, j] + y1 * w1[1, j] + b1[0, j], 0.0)
        z0 = z0 + h * w2[j, 0]
        z1 = z1 + h * w2[j, 1]
    out_ref[...] = jnp.stack([z0 * dv, z1 * dv], axis=0)


def _glue_c_body(sp_ref, g_ref, dinv_ref, b2_ref, out_ref):
    sp = sp_ref[...]
    g = g_ref[...]
    dv = dinv_ref[...]
    b2 = b2_ref[...]
    y0 = dv * (sp[0, 0] + sp[1, 0] + g[0]) + b2[0, 0]
    y1 = dv * (sp[0, 1] + sp[1, 1] + g[1]) + b2[0, 1]
    out_ref[...] = jnp.stack([y0, y1], axis=0)


_PLANAR = jax.ShapeDtypeStruct((2, NPR, 128), jnp.float32)

_glue_a = pl.pallas_call(
    _glue_a_body,
    out_shape=(jax.ShapeDtypeStruct((NPR, 128), jnp.float32), _PLANAR),
)
_glue_b = pl.pallas_call(_glue_b_body, out_shape=_PLANAR)
_glue_c = pl.pallas_call(_glue_c_body, out_shape=_PLANAR)


def _widen(g2):
    return jnp.pad(g2.reshape(2, NP).T, ((0, 0), (0, D - 2)))


def _parts(s):
    return s[:, :, :2].transpose(0, 2, 1).reshape(NC, 2, NPR, 128)


def kernel(x, edge_index, W1, b1, W2, b2):
    ei = edge_index.reshape(2, RB, K, 128)

    zeros8 = jnp.zeros((NP, D), jnp.float32)
    ones8 = jnp.ones((128, D), jnp.float32)

    deg_parts = _sc_degree(ei, ones8, zeros8)
    degp = deg_parts[:, :, 0].reshape(NC, NPR, 128)

    xp = jnp.pad(x, ((0, NP - N), (0, 0)))
    xpl = xp.T.reshape(2, NPR, 128)
    dinv, g1 = _glue_a(degp, xpl)

    s1 = _sc_aggregate(_widen(g1), ei, zeros8)
    g2 = _glue_b(_parts(s1), g1, dinv, W1, b1.reshape(1, 16), W2)
    s2 = _sc_aggregate(_widen(g2), ei, zeros8)
    outp = _glue_c(_parts(s2), g2, dinv, b2.reshape(1, 2))
    return outp.reshape(2, NP).T[:N]
